# baseline (device time: 11607 ns/iter reference)
import jax
import jax.numpy as jnp
from jax import lax
from jax.experimental import pallas as pl
from jax.experimental.pallas import tpu as pltpu

N_DEV = 4


def kernel(x):
    m, n = x.shape
    m_chunk = m // N_DEV

    def body(x_ref, out_ref, rs_recv, ag_recv,
             rs_send_sems, rs_recv_sems, ag_send_sems, ag_recv_sems):
        my = lax.axis_index("i")

        barrier_sem = pltpu.get_barrier_semaphore()
        for k in range(1, N_DEV):
            pl.semaphore_signal(
                barrier_sem, inc=1,
                device_id=((my + k) % N_DEV,),
                device_id_type=pl.DeviceIdType.MESH,
            )
        pl.semaphore_wait(barrier_sem, N_DEV - 1)

        def chunk_rows(c):
            return pl.ds(c * m_chunk, m_chunk)

        rs = []
        for k in range(1, N_DEV):
            peer = (my + k) % N_DEV
            rdma = pltpu.make_async_remote_copy(
                src_ref=x_ref.at[chunk_rows(peer), :],
                dst_ref=rs_recv.at[k - 1],
                send_sem=rs_send_sems.at[k - 1],
                recv_sem=rs_recv_sems.at[k - 1],
                device_id=(peer,),
                device_id_type=pl.DeviceIdType.MESH,
            )
            rdma.start()
            rs.append(rdma)
        rs[0].wait_recv()
        out_ref[chunk_rows(my), :] = x_ref[chunk_rows(my), :] + rs_recv[0]
        rs[2].wait_recv()
        out_ref[chunk_rows(my), :] += rs_recv[2]
        rs[1].wait_recv()
        out_ref[chunk_rows(my), :] += rs_recv[1]

        ag = []
        for k in (2, 1, 3):
            peer = (my + k) % N_DEV
            rdma = pltpu.make_async_remote_copy(
                src_ref=out_ref.at[chunk_rows(my), :],
                dst_ref=ag_recv.at[k - 1],
                send_sem=ag_send_sems.at[k - 1],
                recv_sem=ag_recv_sems.at[k - 1],
                device_id=(peer,),
                device_id_type=pl.DeviceIdType.MESH,
            )
            rdma.start()
            ag.append(rdma)
        for k in (1, 3, 2):
            ag[{2: 0, 1: 1, 3: 2}[k]].wait_recv()
            out_ref[chunk_rows((my - k) % N_DEV), :] = ag_recv[k - 1]

        for rdma in rs:
            rdma.wait_send()
        for rdma in ag:
            rdma.wait_send()

    return pl.pallas_call(
        body,
        out_shape=jax.ShapeDtypeStruct((m, n), x.dtype),
        in_specs=[pl.BlockSpec(memory_space=pltpu.VMEM)],
        out_specs=pl.BlockSpec(memory_space=pltpu.VMEM),
        scratch_shapes=[
            pltpu.VMEM((N_DEV - 1, m_chunk, n), x.dtype),
            pltpu.VMEM((N_DEV - 1, m_chunk, n), x.dtype),
            pltpu.SemaphoreType.DMA((N_DEV - 1,)),
            pltpu.SemaphoreType.DMA((N_DEV - 1,)),
            pltpu.SemaphoreType.DMA((N_DEV - 1,)),
            pltpu.SemaphoreType.DMA((N_DEV - 1,)),
        ],
        compiler_params=pltpu.CompilerParams(collective_id=0),
    )(x)


# device time: 10357 ns/iter; 1.1207x vs baseline; 1.1207x over previous
import jax
import jax.numpy as jnp
from jax import lax
from jax.experimental import pallas as pl
from jax.experimental.pallas import tpu as pltpu

N_DEV = 4


def kernel(x):
    m, n = x.shape
    m_chunk = m // N_DEV

    def body(x_ref, out_ref, rs_recv,
             rs_send_sems, rs_recv_sems, ag_send_sems, ag_recv_sems):
        my = lax.axis_index("i")

        def rows(c):
            return pl.ds(c * m_chunk, m_chunk)

        barrier_sem = pltpu.get_barrier_semaphore()
        for k in range(1, N_DEV):
            pl.semaphore_signal(
                barrier_sem, inc=1,
                device_id=((my + k) % N_DEV,),
                device_id_type=pl.DeviceIdType.MESH,
            )

        rs = []
        for k in range(1, N_DEV):
            peer = (my + k) % N_DEV
            rdma = pltpu.make_async_remote_copy(
                src_ref=x_ref.at[rows(peer), :],
                dst_ref=rs_recv.at[k - 1],
                send_sem=rs_send_sems.at[k - 1],
                recv_sem=rs_recv_sems.at[k - 1],
                device_id=(peer,),
                device_id_type=pl.DeviceIdType.MESH,
            )
            rdma.start()
            rs.append(rdma)

        pl.semaphore_wait(barrier_sem, N_DEV - 1)

        rs[0].wait_recv()
        out_ref[rows(my), :] = x_ref[rows(my), :] + rs_recv[0]
        rs[2].wait_recv()
        out_ref[rows(my), :] += rs_recv[2]
        rs[1].wait_recv()
        out_ref[rows(my), :] += rs_recv[1]

        ag_send = {}
        for k in (2, 1, 3):
            peer = (my + k) % N_DEV
            rdma = pltpu.make_async_remote_copy(
                src_ref=out_ref.at[rows(my), :],
                dst_ref=out_ref.at[rows(my), :],
                send_sem=ag_send_sems.at[k - 1],
                recv_sem=ag_recv_sems.at[k - 1],
                device_id=(peer,),
                device_id_type=pl.DeviceIdType.MESH,
            )
            rdma.start()
            ag_send[k] = rdma
        for k in (1, 3, 2):
            src_dev = (my - k) % N_DEV
            recv = pltpu.make_async_remote_copy(
                src_ref=out_ref.at[rows(my), :],
                dst_ref=out_ref.at[rows(src_dev), :],
                send_sem=ag_send_sems.at[k - 1],
                recv_sem=ag_recv_sems.at[k - 1],
                device_id=((my + k) % N_DEV,),
                device_id_type=pl.DeviceIdType.MESH,
            )
            recv.wait_recv()

        for rdma in rs:
            rdma.wait_send()
        for rdma in ag_send.values():
            rdma.wait_send()

    return pl.pallas_call(
        body,
        out_shape=jax.ShapeDtypeStruct((m, n), x.dtype),
        in_specs=[pl.BlockSpec(memory_space=pltpu.VMEM)],
        out_specs=pl.BlockSpec(memory_space=pltpu.VMEM),
        scratch_shapes=[
            pltpu.VMEM((N_DEV - 1, m_chunk, n), x.dtype),
            pltpu.SemaphoreType.DMA((N_DEV - 1,)),
            pltpu.SemaphoreType.DMA((N_DEV - 1,)),
            pltpu.SemaphoreType.DMA((N_DEV - 1,)),
            pltpu.SemaphoreType.DMA((N_DEV - 1,)),
        ],
        compiler_params=pltpu.CompilerParams(collective_id=0),
    )(x)
